# baseline (speedup 1.0000x reference)
import functools

import jax
import jax.numpy as jnp
from jax import lax
from jax.experimental import pallas as pl
from jax.experimental.pallas import tpu as pltpu
from jax.experimental.pallas import tpu_sc as plsc

B = 16384
D = 64
NC = 2
NS = 16
NW = NC * NS
L = 16
BPW = B // NW
C = 16
NPAIR = BPW // (2 * C)


def _fast_sqrt(x):
    xm = jnp.maximum(x, jnp.float32(1e-30))
    i = plsc.bitcast(xm, jnp.int32)
    i = jnp.int32(0x5F3759DF) - (i >> 1)
    y = plsc.bitcast(i, jnp.float32)
    for _ in range(3):
        y = y * (jnp.float32(1.5) - jnp.float32(0.5) * xm * y * y)
    return xm * y


def _make_sc_call():
    mesh = plsc.VectorSubcoreMesh(core_axis_name="c", subcore_axis_name="s")

    @functools.partial(
        pl.kernel,
        out_type=jax.ShapeDtypeStruct((B,), jnp.float32),
        mesh=mesh,
        compiler_params=pltpu.CompilerParams(
            needs_layout_passes=False, use_tc_tiling_on_sc=True),
        scratch_types=[
            pltpu.VMEM((BPW + 2 * C,), jnp.int32),
            pltpu.VMEM((BPW + 2 * C,), jnp.int32),
            pltpu.VMEM((BPW + 2 * C,), jnp.int32),
            pltpu.VMEM((8 * C, D), jnp.float32),
            pltpu.VMEM((8 * C, D), jnp.float32),
            pltpu.VMEM((8 * C, D), jnp.float32),
            pltpu.VMEM((8 * C, D), jnp.float32),
            pltpu.VMEM((8 * C, 2 * D), jnp.float32),
            pltpu.VMEM((8 * C, 2 * D), jnp.float32),
            pltpu.VMEM((BPW,), jnp.float32),
            pltpu.SemaphoreType.DMA,
            pltpu.SemaphoreType.DMA,
            pltpu.SemaphoreType.DMA,
        ],
    )
    def trans_h(head_hbm, tail_hbm, rel_hbm, ent_hbm, rn_hbm,
                out_hbm, idx_h, idx_t, idx_r, bh0, bh1, bt0, bt1, brn0, brn1,
                score_v, sem0, sem1, semm):
        wid = lax.axis_index("s") * NC + lax.axis_index("c")
        base = pl.multiple_of(wid * BPW, 128)

        cp1 = pltpu.async_copy(head_hbm.at[pl.ds(base, BPW)],
                               idx_h.at[pl.ds(0, BPW)], semm)
        cp2 = pltpu.async_copy(tail_hbm.at[pl.ds(base, BPW)],
                               idx_t.at[pl.ds(0, BPW)], semm)
        cp3 = pltpu.async_copy(rel_hbm.at[pl.ds(base, BPW)],
                               idx_r.at[pl.ds(0, BPW)], semm)
        zpad = jnp.zeros((L,), jnp.int32)
        idx_h[pl.ds(BPW, L)] = zpad
        idx_t[pl.ds(BPW, L)] = zpad
        idx_r[pl.ds(BPW, L)] = zpad
        cp1.wait()
        cp2.wait()
        cp3.wait()

        sets = ((bh0, bt0, brn0, sem0), (bh1, bt1, brn1, sem1))
        lane = lax.iota(jnp.int32, L)
        two = jnp.float32(2.0)
        emax = jnp.int32(1000000 - 1)
        zero = jnp.int32(0)

        def fire(jj, setidx):
            bh, bt, brn, sem = sets[setidx]
            eh = jnp.clip(idx_h[pl.ds(jj * C, L)], zero, emax)
            et = jnp.clip(idx_t[pl.ds(jj * C, L)], zero, emax)
            er = jnp.clip(idx_r[pl.ds(jj * C, L)], zero, jnp.int32(999))
            for l in range(C):
                r0 = pl.multiple_of((eh[l] >> 3) * 8, 8)
                pltpu.async_copy(ent_hbm.at[pl.ds(r0, 8), :],
                                 bh.at[pl.ds(l * 8, 8), :], sem)
                r1 = pl.multiple_of((et[l] >> 3) * 8, 8)
                pltpu.async_copy(ent_hbm.at[pl.ds(r1, 8), :],
                                 bt.at[pl.ds(l * 8, 8), :], sem)
                r2 = pl.multiple_of((er[l] >> 3) * 8, 8)
                pltpu.async_copy(rn_hbm.at[pl.ds(r2, 8), :],
                                 brn.at[pl.ds(l * 8, 8), :], sem)

        def drain(setidx):
            bh, bt, brn, sem = sets[setidx]
            for buf in (bh, bt):
                pltpu.make_async_copy(
                    ent_hbm.at[pl.ds(0, 8 * C), :], buf, sem).wait()
            pltpu.make_async_copy(
                rn_hbm.at[pl.ds(0, 8 * C), :], brn, sem).wait()

        def compute(jj, setidx, lane_off, ev3, sqv):
            bh, bt, brn, _ = sets[setidx]
            eh, et, er = ev3
            for l in range(C):
                ll = lane_off + l
                rh = l * 8 + (eh[ll] & 7)
                rt = l * 8 + (et[ll] & 7)
                rr = l * 8 + (er[ll] & 7)
                pv = jnp.zeros((L,), jnp.float32)
                av = jnp.zeros((L,), jnp.float32)
                cv = jnp.zeros((L,), jnp.float32)
                qv = jnp.zeros((L,), jnp.float32)
                for kk in range(D // L):
                    sl = pl.ds(L * kk, L)
                    h = bh[rh, sl]
                    t = bt[rt, sl]
                    r = brn[rr, sl]
                    n = brn[rr, pl.ds(D + L * kk, L)]
                    u = h - t + r
                    pv = pv + u * u
                    av = av + u * n
                    cv = cv + n * r
                    qv = qv + n * n
                p = jnp.sum(pv)
                a = jnp.sum(av)
                c = jnp.sum(cv)
                q = jnp.sum(qv)
                s = a - c
                sq = p - two * s * a + s * s * q
                sqv = jnp.where(lane == ll, sq, sqv)
            return sqv

        fire(jnp.int32(0), 0)

        def pair(m, _):
            j0 = m * 2
            j1 = j0 + 1
            fire(j1, 1)
            drain(0)
            ev0 = (idx_h[pl.ds(j0 * C, L)], idx_t[pl.ds(j0 * C, L)],
                   idx_r[pl.ds(j0 * C, L)])
            sqv = compute(j0, 0, 0, ev0, jnp.zeros((L,), jnp.float32))
            score_v[pl.ds(j0 * C, L)] = sqv
            fire(j0 + 2, 0)
            drain(1)
            ev1 = (idx_h[pl.ds(j1 * C, L)], idx_t[pl.ds(j1 * C, L)],
                   idx_r[pl.ds(j1 * C, L)])
            sqv = compute(j1, 1, 0, ev1, jnp.zeros((L,), jnp.float32))
            score_v[pl.ds(j1 * C, L)] = sqv
            return 0

        lax.fori_loop(0, NPAIR, pair, 0)
        drain(0)

        def sqrt_pass(g, _):
            score_v[pl.ds(g * L, L)] = _fast_sqrt(score_v[pl.ds(g * L, L)])
            return 0

        lax.fori_loop(0, BPW // L, sqrt_pass, 0)
        pltpu.sync_copy(score_v, out_hbm.at[pl.ds(base, BPW)])

    return trans_h


_sc_call = _make_sc_call()


@jax.jit
def kernel(head, relation, tail, entity_embedding, relation_embedding,
           normal_vector):
    rel_nv = jnp.concatenate([relation_embedding, normal_vector], axis=1)
    return _sc_call(head, tail, relation, entity_embedding, rel_nv)


# speedup vs baseline: 1.0281x; 1.0281x over previous
import functools

import jax
import jax.numpy as jnp
from jax import lax
from jax.experimental import pallas as pl
from jax.experimental.pallas import tpu as pltpu
from jax.experimental.pallas import tpu_sc as plsc

B = 16384
D = 64
NC = 2
NS = 16
NW = NC * NS
L = 16
BPW = B // NW
C = 8
NPAIR = BPW // (2 * C)


def _fast_sqrt(x):
    xm = jnp.maximum(x, jnp.float32(1e-30))
    i = plsc.bitcast(xm, jnp.int32)
    i = jnp.int32(0x5F3759DF) - (i >> 1)
    y = plsc.bitcast(i, jnp.float32)
    for _ in range(3):
        y = y * (jnp.float32(1.5) - jnp.float32(0.5) * xm * y * y)
    return xm * y


def _make_sc_call():
    mesh = plsc.VectorSubcoreMesh(core_axis_name="c", subcore_axis_name="s")

    @functools.partial(
        pl.kernel,
        out_type=jax.ShapeDtypeStruct((B,), jnp.float32),
        mesh=mesh,
        compiler_params=pltpu.CompilerParams(
            needs_layout_passes=False, use_tc_tiling_on_sc=True),
        scratch_types=[
            pltpu.VMEM((BPW + 2 * C,), jnp.int32),
            pltpu.VMEM((BPW + 2 * C,), jnp.int32),
            pltpu.VMEM((BPW + 2 * C,), jnp.int32),
            pltpu.VMEM((8 * C, D), jnp.float32),
            pltpu.VMEM((8 * C, D), jnp.float32),
            pltpu.VMEM((8 * C, D), jnp.float32),
            pltpu.VMEM((8 * C, D), jnp.float32),
            pltpu.VMEM((8 * C, 2 * D), jnp.float32),
            pltpu.VMEM((8 * C, 2 * D), jnp.float32),
            pltpu.VMEM((BPW,), jnp.float32),
            pltpu.SemaphoreType.DMA,
            pltpu.SemaphoreType.DMA,
            pltpu.SemaphoreType.DMA,
        ],
    )
    def trans_h(head_hbm, tail_hbm, rel_hbm, ent_hbm, rn_hbm,
                out_hbm, idx_h, idx_t, idx_r, bh0, bh1, bt0, bt1, brn0, brn1,
                score_v, sem0, sem1, semm):
        wid = lax.axis_index("s") * NC + lax.axis_index("c")
        base = pl.multiple_of(wid * BPW, 128)

        cp1 = pltpu.async_copy(head_hbm.at[pl.ds(base, BPW)],
                               idx_h.at[pl.ds(0, BPW)], semm)
        cp2 = pltpu.async_copy(tail_hbm.at[pl.ds(base, BPW)],
                               idx_t.at[pl.ds(0, BPW)], semm)
        cp3 = pltpu.async_copy(rel_hbm.at[pl.ds(base, BPW)],
                               idx_r.at[pl.ds(0, BPW)], semm)
        zpad = jnp.zeros((L,), jnp.int32)
        idx_h[pl.ds(BPW, L)] = zpad
        idx_t[pl.ds(BPW, L)] = zpad
        idx_r[pl.ds(BPW, L)] = zpad
        cp1.wait()
        cp2.wait()
        cp3.wait()

        sets = ((bh0, bt0, brn0, sem0), (bh1, bt1, brn1, sem1))
        lane = lax.iota(jnp.int32, L)
        two = jnp.float32(2.0)
        emax = jnp.int32(1000000 - 1)
        zero = jnp.int32(0)

        def fire(jj, setidx):
            bh, bt, brn, sem = sets[setidx]
            eh = jnp.clip(idx_h[pl.ds(jj * C, L)], zero, emax)
            et = jnp.clip(idx_t[pl.ds(jj * C, L)], zero, emax)
            er = jnp.clip(idx_r[pl.ds(jj * C, L)], zero, jnp.int32(999))
            for l in range(C):
                r0 = pl.multiple_of((eh[l] >> 3) * 8, 8)
                pltpu.async_copy(ent_hbm.at[pl.ds(r0, 8), :],
                                 bh.at[pl.ds(l * 8, 8), :], sem)
                r1 = pl.multiple_of((et[l] >> 3) * 8, 8)
                pltpu.async_copy(ent_hbm.at[pl.ds(r1, 8), :],
                                 bt.at[pl.ds(l * 8, 8), :], sem)
                r2 = pl.multiple_of((er[l] >> 3) * 8, 8)
                pltpu.async_copy(rn_hbm.at[pl.ds(r2, 8), :],
                                 brn.at[pl.ds(l * 8, 8), :], sem)

        def drain(setidx):
            bh, bt, brn, sem = sets[setidx]
            for buf in (bh, bt):
                pltpu.make_async_copy(
                    ent_hbm.at[pl.ds(0, 8 * C), :], buf, sem).wait()
            pltpu.make_async_copy(
                rn_hbm.at[pl.ds(0, 8 * C), :], brn, sem).wait()

        def compute(jj, setidx, lane_off, ev3, sqv):
            bh, bt, brn, _ = sets[setidx]
            eh, et, er = ev3
            for l in range(C):
                ll = lane_off + l
                rh = l * 8 + (eh[ll] & 7)
                rt = l * 8 + (et[ll] & 7)
                rr = l * 8 + (er[ll] & 7)
                pv = jnp.zeros((L,), jnp.float32)
                av = jnp.zeros((L,), jnp.float32)
                cv = jnp.zeros((L,), jnp.float32)
                qv = jnp.zeros((L,), jnp.float32)
                for kk in range(D // L):
                    sl = pl.ds(L * kk, L)
                    h = bh[rh, sl]
                    t = bt[rt, sl]
                    r = brn[rr, sl]
                    n = brn[rr, pl.ds(D + L * kk, L)]
                    u = h - t + r
                    pv = pv + u * u
                    av = av + u * n
                    cv = cv + n * r
                    qv = qv + n * n
                p = jnp.sum(pv)
                a = jnp.sum(av)
                c = jnp.sum(cv)
                q = jnp.sum(qv)
                s = a - c
                sq = p - two * s * a + s * s * q
                sqv = jnp.where(lane == ll, sq, sqv)
            return sqv

        fire(jnp.int32(0), 0)

        def pair(m, _):
            j0 = m * 2
            j1 = j0 + 1
            gbase = m * L
            eh = idx_h[pl.ds(gbase, L)]
            et = idx_t[pl.ds(gbase, L)]
            er = idx_r[pl.ds(gbase, L)]
            ev3 = (eh, et, er)
            fire(j1, 1)
            drain(0)
            sqv = compute(j0, 0, 0, ev3, jnp.zeros((L,), jnp.float32))
            fire(j0 + 2, 0)
            drain(1)
            sqv = compute(j1, 1, C, ev3, sqv)
            score_v[pl.ds(gbase, L)] = sqv
            return 0

        lax.fori_loop(0, NPAIR, pair, 0)
        drain(0)

        def sqrt_pass(g, _):
            score_v[pl.ds(g * L, L)] = _fast_sqrt(score_v[pl.ds(g * L, L)])
            return 0

        lax.fori_loop(0, BPW // L, sqrt_pass, 0)
        pltpu.sync_copy(score_v, out_hbm.at[pl.ds(base, BPW)])

    return trans_h


_sc_call = _make_sc_call()


@jax.jit
def kernel(head, relation, tail, entity_embedding, relation_embedding,
           normal_vector):
    rel_nv = jnp.concatenate([relation_embedding, normal_vector], axis=1)
    return _sc_call(head, tail, relation, entity_embedding, rel_nv)
